# baseline (speedup 1.0000x reference)
import jax
import jax.numpy as jnp
from jax.experimental import pallas as pl
from jax.experimental.pallas import tpu as pltpu

_B = 64
_N = 100000
_BLK = 30720
_NB = (_N + _BLK - 1) // _BLK
_SEG = 32


def _fused_body(p_ref, out_ref, max_ref, idx_ref):
    j = pl.program_id(0)

    @pl.when(j == 0)
    def _init():
        max_ref[...] = jnp.full((_B,), -jnp.inf, jnp.float32)
        idx_ref[...] = jnp.zeros((_B,), jnp.int32)

    x = p_ref[...]
    col = jax.lax.broadcasted_iota(jnp.int32, (_B, _BLK), 1) + j * _BLK

    def _update(xv):
        blk_max = jnp.max(xv, axis=1)
        is_max = xv == blk_max[:, None]
        blk_idx = jnp.min(jnp.where(is_max, col, _N), axis=1)
        better = blk_max > max_ref[...]
        max_ref[...] = jnp.where(better, blk_max, max_ref[...])
        idx_ref[...] = jnp.where(better, blk_idx, idx_ref[...])

    @pl.when(j < _NB - 1)
    def _main():
        _update(x)
        out_ref[...] = jnp.zeros((_B, _BLK), jnp.float32)

    @pl.when(j == _NB - 1)
    def _last():
        _update(jnp.where(col < _N, x, -jnp.inf))
        out_ref[...] = (col == idx_ref[...][:, None]).astype(jnp.float32)


def _fixup_body(t_ref, aidx_ref, in_ref, out_ref, buf_ref, sem):
    del in_ref
    colio = jax.lax.broadcasted_iota(jnp.int32, (8, 128), 1)
    copies = []
    for r in range(_B):
        g = r // 8
        t_r = t_ref[r]
        hit = aidx_ref[8 * g : 8 * g + 8, :] == t_r * 128 + colio
        buf_ref[8 * r : 8 * r + 8, :] = hit.astype(jnp.float32)
        cp = pltpu.make_async_copy(
            buf_ref.at[pl.ds(8 * r, 8), :],
            out_ref.at[pl.ds(8 * g, 8), pl.ds(pl.multiple_of(t_r * 128, 128), 128)],
            sem,
        )
        cp.start()
        copies.append(cp)
    for cp in copies:
        cp.wait()


def kernel(state_vector, probabilities):
    del state_vector

    collapsed0, max_val, arg_idx = pl.pallas_call(
        _fused_body,
        grid=(_NB,),
        in_specs=[pl.BlockSpec((_B, _BLK), lambda j: (0, j))],
        out_specs=[
            pl.BlockSpec((_B, _BLK), lambda j: (0, j)),
            pl.BlockSpec((_B,), lambda j: (0,)),
            pl.BlockSpec((_B,), lambda j: (0,)),
        ],
        out_shape=[
            jax.ShapeDtypeStruct((_B, _N), jnp.float32),
            jax.ShapeDtypeStruct((_B,), jnp.float32),
            jax.ShapeDtypeStruct((_B,), jnp.int32),
        ],
    )(probabilities)

    tile_idx = jnp.minimum(arg_idx // 128, (_NB - 1) * _BLK // 128 - 1)
    aidx = jnp.broadcast_to(arg_idx[:, None], (_B, 128))

    collapsed = pl.pallas_call(
        _fixup_body,
        in_specs=[
            pl.BlockSpec(memory_space=pltpu.SMEM),
            pl.BlockSpec(memory_space=pltpu.VMEM),
            pl.BlockSpec(memory_space=pl.ANY),
        ],
        out_specs=pl.BlockSpec(memory_space=pl.ANY),
        out_shape=jax.ShapeDtypeStruct((_B, _N), jnp.float32),
        scratch_shapes=[
            pltpu.VMEM((8 * _B, 128), jnp.float32),
            pltpu.SemaphoreType.DMA,
        ],
        input_output_aliases={2: 0},
    )(tile_idx, aidx, collapsed0)

    return collapsed, max_val


# speedup vs baseline: 1.0022x; 1.0022x over previous
import jax
import jax.numpy as jnp
from jax.experimental import pallas as pl
from jax.experimental.pallas import tpu as pltpu

_B = 64
_N = 100000
_BLK = 36864
_NB = (_N + _BLK - 1) // _BLK
_SEG = 32


def _fused_body(p_ref, out_ref, max_ref, idx_ref):
    j = pl.program_id(0)

    @pl.when(j == 0)
    def _init():
        max_ref[...] = jnp.full((_B,), -jnp.inf, jnp.float32)
        idx_ref[...] = jnp.zeros((_B,), jnp.int32)

    x = p_ref[...]
    col = jax.lax.broadcasted_iota(jnp.int32, (_B, _BLK), 1) + j * _BLK

    def _update(xv):
        blk_max = jnp.max(xv, axis=1)
        is_max = xv == blk_max[:, None]
        blk_idx = jnp.min(jnp.where(is_max, col, _N), axis=1)
        better = blk_max > max_ref[...]
        max_ref[...] = jnp.where(better, blk_max, max_ref[...])
        idx_ref[...] = jnp.where(better, blk_idx, idx_ref[...])

    @pl.when(j < _NB - 1)
    def _main():
        _update(x)
        out_ref[...] = jnp.zeros((_B, _BLK), jnp.float32)

    @pl.when(j == _NB - 1)
    def _last():
        _update(jnp.where(col < _N, x, -jnp.inf))
        out_ref[...] = (col == idx_ref[...][:, None]).astype(jnp.float32)


def _fixup_body(t_ref, aidx_ref, in_ref, out_ref, buf_ref, sem):
    del in_ref
    colio = jax.lax.broadcasted_iota(jnp.int32, (8, 128), 1)
    copies = []
    for r in range(_B):
        g = r // 8
        t_r = t_ref[r]
        hit = aidx_ref[8 * g : 8 * g + 8, :] == t_r * 128 + colio
        buf_ref[8 * r : 8 * r + 8, :] = hit.astype(jnp.float32)
        cp = pltpu.make_async_copy(
            buf_ref.at[pl.ds(8 * r, 8), :],
            out_ref.at[pl.ds(8 * g, 8), pl.ds(pl.multiple_of(t_r * 128, 128), 128)],
            sem,
        )
        cp.start()
        copies.append(cp)
    for cp in copies:
        cp.wait()


def kernel(state_vector, probabilities):
    del state_vector

    collapsed0, max_val, arg_idx = pl.pallas_call(
        _fused_body,
        grid=(_NB,),
        in_specs=[pl.BlockSpec((_B, _BLK), lambda j: (0, j))],
        out_specs=[
            pl.BlockSpec((_B, _BLK), lambda j: (0, j)),
            pl.BlockSpec((_B,), lambda j: (0,)),
            pl.BlockSpec((_B,), lambda j: (0,)),
        ],
        out_shape=[
            jax.ShapeDtypeStruct((_B, _N), jnp.float32),
            jax.ShapeDtypeStruct((_B,), jnp.float32),
            jax.ShapeDtypeStruct((_B,), jnp.int32),
        ],
    )(probabilities)

    tile_idx = jnp.minimum(arg_idx // 128, (_NB - 1) * _BLK // 128 - 1)
    aidx = jnp.broadcast_to(arg_idx[:, None], (_B, 128))

    collapsed = pl.pallas_call(
        _fixup_body,
        in_specs=[
            pl.BlockSpec(memory_space=pltpu.SMEM),
            pl.BlockSpec(memory_space=pltpu.VMEM),
            pl.BlockSpec(memory_space=pl.ANY),
        ],
        out_specs=pl.BlockSpec(memory_space=pl.ANY),
        out_shape=jax.ShapeDtypeStruct((_B, _N), jnp.float32),
        scratch_shapes=[
            pltpu.VMEM((8 * _B, 128), jnp.float32),
            pltpu.SemaphoreType.DMA,
        ],
        input_output_aliases={2: 0},
    )(tile_idx, aidx, collapsed0)

    return collapsed, max_val


# speedup vs baseline: 1.0088x; 1.0065x over previous
import jax
import jax.numpy as jnp
from jax.experimental import pallas as pl
from jax.experimental.pallas import tpu as pltpu

_B = 64
_N = 100000
_BLK = 28672
_NB = (_N + _BLK - 1) // _BLK


def _fused_body(p_ref, out_ref, max_ref, idx_ref):
    j = pl.program_id(0)

    @pl.when(j == 0)
    def _init():
        max_ref[...] = jnp.full((_B,), -jnp.inf, jnp.float32)
        idx_ref[...] = jnp.zeros((_B,), jnp.int32)

    x = p_ref[...]
    col = jax.lax.broadcasted_iota(jnp.int32, (_B, _BLK), 1) + j * _BLK

    def _update(xv):
        blk_max = jnp.max(xv, axis=1)
        is_max = xv == blk_max[:, None]
        blk_idx = jnp.min(jnp.where(is_max, col, _N), axis=1)
        better = blk_max > max_ref[...]
        max_ref[...] = jnp.where(better, blk_max, max_ref[...])
        idx_ref[...] = jnp.where(better, blk_idx, idx_ref[...])

    @pl.when(j < _NB - 1)
    def _main():
        _update(x)
        out_ref[...] = jnp.zeros((_B, _BLK), jnp.float32)

    @pl.when(j == _NB - 1)
    def _last():
        _update(jnp.where(col < _N, x, -jnp.inf))
        out_ref[...] = (col == idx_ref[...][:, None]).astype(jnp.float32)


def _fixup_body(t_ref, aidx_ref, in_ref, out_ref, buf_ref, sem):
    del in_ref
    colio = jax.lax.broadcasted_iota(jnp.int32, (8, 128), 1)
    copies = []
    for r in range(_B):
        g = r // 8
        t_r = t_ref[r]
        hit = aidx_ref[8 * g : 8 * g + 8, :] == t_r * 128 + colio
        buf_ref[8 * r : 8 * r + 8, :] = hit.astype(jnp.float32)
        cp = pltpu.make_async_copy(
            buf_ref.at[pl.ds(8 * r, 8), :],
            out_ref.at[pl.ds(8 * g, 8), pl.ds(pl.multiple_of(t_r * 128, 128), 128)],
            sem,
        )
        cp.start()
        copies.append(cp)
    for cp in copies:
        cp.wait()


def kernel(state_vector, probabilities):
    del state_vector

    collapsed0, max_val, arg_idx = pl.pallas_call(
        _fused_body,
        grid=(_NB,),
        in_specs=[pl.BlockSpec((_B, _BLK), lambda j: (0, j))],
        out_specs=[
            pl.BlockSpec((_B, _BLK), lambda j: (0, j)),
            pl.BlockSpec((_B,), lambda j: (0,)),
            pl.BlockSpec((_B,), lambda j: (0,)),
        ],
        out_shape=[
            jax.ShapeDtypeStruct((_B, _N), jnp.float32),
            jax.ShapeDtypeStruct((_B,), jnp.float32),
            jax.ShapeDtypeStruct((_B,), jnp.int32),
        ],
    )(probabilities)

    tile_idx = jnp.minimum(arg_idx // 128, (_NB - 1) * _BLK // 128 - 1)
    aidx = jnp.broadcast_to(arg_idx[:, None], (_B, 128))

    collapsed = pl.pallas_call(
        _fixup_body,
        in_specs=[
            pl.BlockSpec(memory_space=pltpu.SMEM),
            pl.BlockSpec(memory_space=pltpu.VMEM),
            pl.BlockSpec(memory_space=pl.ANY),
        ],
        out_specs=pl.BlockSpec(memory_space=pl.ANY),
        out_shape=jax.ShapeDtypeStruct((_B, _N), jnp.float32),
        scratch_shapes=[
            pltpu.VMEM((8 * _B, 128), jnp.float32),
            pltpu.SemaphoreType.DMA,
        ],
        input_output_aliases={2: 0},
    )(tile_idx, aidx, collapsed0)

    return collapsed, max_val
